# baseline (speedup 1.0000x reference)
import functools

import jax
import jax.numpy as jnp
from jax import lax
from jax.experimental import pallas as pl
from jax.experimental.pallas import tpu as pltpu
from jax.experimental.pallas import tpu_sc as plsc

F = 16
SPC = 16
NC = 2
NW = NC * SPC
CHUNK = 128
CH2 = 2048


def _propagate_T(hsT, sidx, didx, np_, nch):
  half = nch * CH2

  mesh = plsc.VectorSubcoreMesh(core_axis_name="c", subcore_axis_name="s")

  @functools.partial(
      pl.kernel,
      mesh=mesh,
      out_type=[
          jax.ShapeDtypeStruct((F, np_), jnp.float32),
          jax.ShapeDtypeStruct((F, np_), jnp.float32),
      ],
      scratch_types=[
          pltpu.VMEM((np_,), jnp.float32),
          pltpu.VMEM((np_,), jnp.float32),
          pltpu.VMEM((2 * CH2,), jnp.int32),
          pltpu.VMEM((2 * CH2,), jnp.int32),
          pltpu.SemaphoreType.DMA,
      ],
      compiler_params=pltpu.CompilerParams(
          use_tc_tiling_on_sc=False, needs_layout_passes=False),
  )
  def k(hsT_hbm, sidx_hbm, didx_hbm, paT_hbm, pbT_hbm, ht, acc, sv, dv, sem):
    c = lax.axis_index("c")
    s = lax.axis_index("s")
    base = c * half

    tcp = pltpu.async_copy(hsT_hbm.at[s], ht, sem)

    def zbody(i, carry):
      acc[pl.ds(i * F, F)] = jnp.zeros((F,), jnp.float32)
      return carry

    lax.fori_loop(0, np_ // F, zbody, 0)
    tcp.wait()

    pltpu.async_copy(sidx_hbm.at[pl.ds(base, CH2)], sv.at[pl.ds(0, CH2)], sem)
    pltpu.async_copy(didx_hbm.at[pl.ds(base, CH2)], dv.at[pl.ds(0, CH2)], sem)

    def chunk(j, carry):
      off = lax.rem(j, 2) * CH2
      nxt = CH2 - off
      pltpu.make_async_copy(
          sidx_hbm.at[pl.ds(base + j * CH2, CH2)],
          sv.at[pl.ds(off, CH2)], sem).wait()
      pltpu.make_async_copy(
          didx_hbm.at[pl.ds(base + j * CH2, CH2)],
          dv.at[pl.ds(off, CH2)], sem).wait()

      @pl.when(j + 1 < nch)
      def _():
        pltpu.async_copy(
            sidx_hbm.at[pl.ds(base + (j + 1) * CH2, CH2)],
            sv.at[pl.ds(nxt, CH2)], sem)
        pltpu.async_copy(
            didx_hbm.at[pl.ds(base + (j + 1) * CH2, CH2)],
            dv.at[pl.ds(nxt, CH2)], sem)

      def inner(v, carry2):
        p = off + v * F
        s16 = sv[pl.ds(p, F)]
        d16 = dv[pl.ds(p, F)]
        vals = plsc.load_gather(ht, [s16])
        plsc.addupdate_scatter(acc, [d16], vals)
        return carry2

      lax.fori_loop(0, CH2 // F, inner, 0)
      return carry

    lax.fori_loop(0, nch, chunk, 0)

    @pl.when(c == 0)
    def _():
      pltpu.sync_copy(acc, paT_hbm.at[s])

    @pl.when(c == 1)
    def _():
      pltpu.sync_copy(acc, pbT_hbm.at[s])

  return k(hsT, sidx, didx)


def _degrees(didx2, np_, ch_w):
  rps = np_ // SPC

  mesh = plsc.VectorSubcoreMesh(core_axis_name="c", subcore_axis_name="s")

  @functools.partial(
      pl.kernel,
      mesh=mesh,
      out_type=[
          jax.ShapeDtypeStruct((np_,), jnp.float32),
          jax.ShapeDtypeStruct((np_,), jnp.float32),
      ],
      scratch_types=[
          pltpu.VMEM_SHARED((np_,), jnp.float32),
          pltpu.VMEM((ch_w, CHUNK), jnp.int32),
          pltpu.VMEM((rps,), jnp.float32),
      ],
      compiler_params=pltpu.CompilerParams(use_tc_tiling_on_sc=False),
  )
  def k(didx_hbm, da_hbm, db_hbm, accd, dv, buf):
    c = lax.axis_index("c")
    s = lax.axis_index("s")
    w = c * SPC + s
    r0 = s * rps
    for i in range(rps // F):
      buf[pl.ds(i * F, F)] = jnp.full((F,), 1.0, jnp.float32)
    pltpu.sync_copy(buf, accd.at[pl.ds(r0, rps)])
    pltpu.sync_copy(didx_hbm.at[pl.ds(w * ch_w, ch_w)], dv)
    plsc.subcore_barrier()

    def body(j, carry):
      pltpu.sync_copy(buf.at[pl.ds(0, CHUNK)], accd.at[dv.at[j]], add=True)
      return carry

    lax.fori_loop(0, ch_w, body, 0)

    plsc.subcore_barrier()

    @pl.when(c == 0)
    def _():
      pltpu.sync_copy(accd.at[pl.ds(r0, rps)], da_hbm.at[pl.ds(r0, rps)])

    @pl.when(c == 1)
    def _():
      pltpu.sync_copy(accd.at[pl.ds(r0, rps)], db_hbm.at[pl.ds(r0, rps)])

  return k(didx2)


def _tc_prep(da, db, x, w1, np_):
  def body(da_ref, db_ref, x_ref, w1_ref, dinv_ref, hsT_ref):
    deg = da_ref[...] + db_ref[...] - 1.0
    dinv = lax.rsqrt(deg)
    hT = lax.dot_general(
        w1_ref[...], x_ref[...],
        dimension_numbers=(((0,), (1,)), ((), ())),
        preferred_element_type=jnp.float32)
    dinv_ref[...] = dinv
    hsT_ref[...] = dinv * hT

  return pl.pallas_call(
      body,
      out_shape=[
          jax.ShapeDtypeStruct((1, np_), jnp.float32),
          jax.ShapeDtypeStruct((F, np_), jnp.float32),
      ],
  )(da, db, x, w1)


def _tc_layer(dinv, paT, pbT, hsT, b1c, w2pT, np_):
  def body(dinv_ref, pa_ref, pb_ref, hsp_ref, b1_ref, w2t_ref, out_ref):
    t = dinv_ref[...] * (pa_ref[...] + pb_ref[...] + hsp_ref[...]) + b1_ref[...]
    h2 = jnp.maximum(t, 0.0)
    out_ref[...] = dinv_ref[...] * jnp.dot(
        w2t_ref[...], h2, preferred_element_type=jnp.float32)

  return pl.pallas_call(
      body,
      out_shape=jax.ShapeDtypeStruct((F, np_), jnp.float32),
  )(dinv, paT, pbT, hsT, b1c, w2pT)


def _tc_final(dinv, qaT, qbT, hs2T, b2c, np_, c_):
  def body(dinv_ref, qa_ref, qb_ref, hsp_ref, b2_ref, out_ref):
    t = dinv_ref[...] * (qa_ref[...] + qb_ref[...] + hsp_ref[...]) + b2_ref[...]
    r = jnp.maximum(t, 0.0)
    row = lax.broadcasted_iota(jnp.int32, (F, np_), 0)
    valid = row < c_
    rm = jnp.where(valid, r, jnp.float32(-1e30))
    m = jnp.max(rm, axis=0, keepdims=True)
    e = jnp.where(valid, jnp.exp(rm - m), 0.0)
    ssum = jnp.sum(e, axis=0, keepdims=True)
    res = rm - m - jnp.log(ssum)
    out_ref[...] = res.T

  return pl.pallas_call(
      body,
      out_shape=jax.ShapeDtypeStruct((np_, F), jnp.float32),
  )(dinv, qaT, qbT, hs2T, b2c)


def kernel(x, edge_index, W1, b1, W2, b2):
  n, d = x.shape
  h = W1.shape[1]
  c_ = W2.shape[1]
  assert h == F
  e = edge_index.shape[1]

  np_ = ((n + 255) // 256) * 256
  nch = -(-e // (NC * CH2))
  ep = NC * nch * CH2
  ch_w = ep // (NW * CHUNK)

  src = edge_index[0]
  dst = edge_index[1]
  pad = jnp.full((ep - e,), n, dtype=jnp.int32)
  sidx = jnp.concatenate([src, pad])
  didx = jnp.concatenate([dst, pad])

  xp = jnp.pad(x, ((0, np_ - n), (0, 0)))

  da, db = _degrees(didx.reshape(NW * ch_w, CHUNK), np_, ch_w)
  dinv, hsT = _tc_prep(da.reshape(1, np_), db.reshape(1, np_), xp, W1, np_)
  paT, pbT = _propagate_T(hsT, sidx, didx, np_, nch)
  w2pT = jnp.pad(W2, ((0, 0), (0, F - c_))).T
  hs2T = _tc_layer(dinv, paT, pbT, hsT, b1.reshape(F, 1), w2pT, np_)
  qaT, qbT = _propagate_T(hs2T, sidx, didx, np_, nch)
  b2c = jnp.pad(b2, (0, F - c_)).reshape(F, 1)
  out = _tc_final(dinv, qaT, qbT, hs2T, b2c, np_, c_)
  return out[:n, :c_]


# speedup vs baseline: 1.0341x; 1.0341x over previous
import functools

import jax
import jax.numpy as jnp
from jax import lax
from jax.experimental import pallas as pl
from jax.experimental.pallas import tpu as pltpu
from jax.experimental.pallas import tpu_sc as plsc

F = 16
SPC = 16
NC = 2
NW = NC * SPC
CHUNK = 128
CH2 = 2048


def _propagate_T(hsT, sidx, didx, np_, nch):
  half = nch * CH2

  mesh = plsc.VectorSubcoreMesh(core_axis_name="c", subcore_axis_name="s")

  @functools.partial(
      pl.kernel,
      mesh=mesh,
      out_type=[
          jax.ShapeDtypeStruct((F, np_), jnp.float32),
          jax.ShapeDtypeStruct((F, np_), jnp.float32),
      ],
      scratch_types=[
          pltpu.VMEM((np_,), jnp.float32),
          pltpu.VMEM((np_,), jnp.float32),
          pltpu.VMEM((2 * CH2,), jnp.int32),
          pltpu.VMEM((2 * CH2,), jnp.int32),
          pltpu.SemaphoreType.DMA,
      ],
      compiler_params=pltpu.CompilerParams(
          use_tc_tiling_on_sc=False, needs_layout_passes=False),
  )
  def k(hsT_hbm, sidx_hbm, didx_hbm, paT_hbm, pbT_hbm, ht, acc, sv, dv, sem):
    c = lax.axis_index("c")
    s = lax.axis_index("s")
    base = c * half

    tcp = pltpu.async_copy(hsT_hbm.at[s], ht, sem)

    def zbody(i, carry):
      for u in range(8):
        acc[pl.ds((i * 8 + u) * F, F)] = jnp.zeros((F,), jnp.float32)
      return carry

    lax.fori_loop(0, np_ // F // 8, zbody, 0)
    tcp.wait()

    pltpu.async_copy(sidx_hbm.at[pl.ds(base, CH2)], sv.at[pl.ds(0, CH2)], sem)
    pltpu.async_copy(didx_hbm.at[pl.ds(base, CH2)], dv.at[pl.ds(0, CH2)], sem)

    def chunk(j, carry):
      off = lax.rem(j, 2) * CH2
      nxt = CH2 - off
      pltpu.make_async_copy(
          sidx_hbm.at[pl.ds(base + j * CH2, CH2)],
          sv.at[pl.ds(off, CH2)], sem).wait()
      pltpu.make_async_copy(
          didx_hbm.at[pl.ds(base + j * CH2, CH2)],
          dv.at[pl.ds(off, CH2)], sem).wait()

      @pl.when(j + 1 < nch)
      def _():
        pltpu.async_copy(
            sidx_hbm.at[pl.ds(base + (j + 1) * CH2, CH2)],
            sv.at[pl.ds(nxt, CH2)], sem)
        pltpu.async_copy(
            didx_hbm.at[pl.ds(base + (j + 1) * CH2, CH2)],
            dv.at[pl.ds(nxt, CH2)], sem)

      def inner(v, carry2):
        for u in range(8):
          p = off + (v * 8 + u) * F
          s16 = sv[pl.ds(p, F)]
          d16 = dv[pl.ds(p, F)]
          vals = plsc.load_gather(ht, [s16])
          plsc.addupdate_scatter(acc, [d16], vals)
        return carry2

      lax.fori_loop(0, CH2 // F // 8, inner, 0)
      return carry

    lax.fori_loop(0, nch, chunk, 0)

    @pl.when(c == 0)
    def _():
      pltpu.sync_copy(acc, paT_hbm.at[s])

    @pl.when(c == 1)
    def _():
      pltpu.sync_copy(acc, pbT_hbm.at[s])

  return k(hsT, sidx, didx)


def _degrees(didx2, np_, ch_w):
  rps = np_ // SPC

  mesh = plsc.VectorSubcoreMesh(core_axis_name="c", subcore_axis_name="s")

  @functools.partial(
      pl.kernel,
      mesh=mesh,
      out_type=[
          jax.ShapeDtypeStruct((np_,), jnp.float32),
          jax.ShapeDtypeStruct((np_,), jnp.float32),
      ],
      scratch_types=[
          pltpu.VMEM_SHARED((np_,), jnp.float32),
          pltpu.VMEM((ch_w, CHUNK), jnp.int32),
          pltpu.VMEM((rps,), jnp.float32),
      ],
      compiler_params=pltpu.CompilerParams(use_tc_tiling_on_sc=False),
  )
  def k(didx_hbm, da_hbm, db_hbm, accd, dv, buf):
    c = lax.axis_index("c")
    s = lax.axis_index("s")
    w = c * SPC + s
    r0 = s * rps
    for i in range(rps // F):
      buf[pl.ds(i * F, F)] = jnp.full((F,), 1.0, jnp.float32)
    pltpu.sync_copy(buf, accd.at[pl.ds(r0, rps)])
    pltpu.sync_copy(didx_hbm.at[pl.ds(w * ch_w, ch_w)], dv)
    plsc.subcore_barrier()

    def body(j, carry):
      pltpu.sync_copy(buf.at[pl.ds(0, CHUNK)], accd.at[dv.at[j]], add=True)
      return carry

    lax.fori_loop(0, ch_w, body, 0)

    plsc.subcore_barrier()

    @pl.when(c == 0)
    def _():
      pltpu.sync_copy(accd.at[pl.ds(r0, rps)], da_hbm.at[pl.ds(r0, rps)])

    @pl.when(c == 1)
    def _():
      pltpu.sync_copy(accd.at[pl.ds(r0, rps)], db_hbm.at[pl.ds(r0, rps)])

  return k(didx2)


def _tc_prep(da, db, x, w1, np_):
  def body(da_ref, db_ref, x_ref, w1_ref, dinv_ref, hsT_ref):
    deg = da_ref[...] + db_ref[...] - 1.0
    dinv = lax.rsqrt(deg)
    hT = lax.dot_general(
        w1_ref[...], x_ref[...],
        dimension_numbers=(((0,), (1,)), ((), ())),
        preferred_element_type=jnp.float32)
    dinv_ref[...] = dinv
    hsT_ref[...] = dinv * hT

  return pl.pallas_call(
      body,
      out_shape=[
          jax.ShapeDtypeStruct((1, np_), jnp.float32),
          jax.ShapeDtypeStruct((F, np_), jnp.float32),
      ],
  )(da, db, x, w1)


def _tc_layer(dinv, paT, pbT, hsT, b1c, w2pT, np_):
  def body(dinv_ref, pa_ref, pb_ref, hsp_ref, b1_ref, w2t_ref, out_ref):
    t = dinv_ref[...] * (pa_ref[...] + pb_ref[...] + hsp_ref[...]) + b1_ref[...]
    h2 = jnp.maximum(t, 0.0)
    out_ref[...] = dinv_ref[...] * jnp.dot(
        w2t_ref[...], h2, preferred_element_type=jnp.float32)

  return pl.pallas_call(
      body,
      out_shape=jax.ShapeDtypeStruct((F, np_), jnp.float32),
  )(dinv, paT, pbT, hsT, b1c, w2pT)


def _tc_final(dinv, qaT, qbT, hs2T, b2c, np_, c_):
  def body(dinv_ref, qa_ref, qb_ref, hsp_ref, b2_ref, out_ref):
    t = dinv_ref[...] * (qa_ref[...] + qb_ref[...] + hsp_ref[...]) + b2_ref[...]
    r = jnp.maximum(t, 0.0)
    row = lax.broadcasted_iota(jnp.int32, (F, np_), 0)
    valid = row < c_
    rm = jnp.where(valid, r, jnp.float32(-1e30))
    m = jnp.max(rm, axis=0, keepdims=True)
    e = jnp.where(valid, jnp.exp(rm - m), 0.0)
    ssum = jnp.sum(e, axis=0, keepdims=True)
    res = rm - m - jnp.log(ssum)
    out_ref[...] = res.T

  return pl.pallas_call(
      body,
      out_shape=jax.ShapeDtypeStruct((np_, F), jnp.float32),
  )(dinv, qaT, qbT, hs2T, b2c)


def kernel(x, edge_index, W1, b1, W2, b2):
  n, d = x.shape
  h = W1.shape[1]
  c_ = W2.shape[1]
  assert h == F
  e = edge_index.shape[1]

  np_ = ((n + 255) // 256) * 256
  nch = -(-e // (NC * CH2))
  ep = NC * nch * CH2
  ch_w = ep // (NW * CHUNK)

  src = edge_index[0]
  dst = edge_index[1]
  pad = jnp.full((ep - e,), n, dtype=jnp.int32)
  sidx = jnp.concatenate([src, pad])
  didx = jnp.concatenate([dst, pad])

  xp = jnp.pad(x, ((0, np_ - n), (0, 0)))

  da, db = _degrees(didx.reshape(NW * ch_w, CHUNK), np_, ch_w)
  dinv, hsT = _tc_prep(da.reshape(1, np_), db.reshape(1, np_), xp, W1, np_)
  paT, pbT = _propagate_T(hsT, sidx, didx, np_, nch)
  w2pT = jnp.pad(W2, ((0, 0), (0, F - c_))).T
  hs2T = _tc_layer(dinv, paT, pbT, hsT, b1.reshape(F, 1), w2pT, np_)
  qaT, qbT = _propagate_T(hs2T, sidx, didx, np_, nch)
  b2c = jnp.pad(b2, (0, F - c_)).reshape(F, 1)
  out = _tc_final(dinv, qaT, qbT, hs2T, b2c, np_, c_)
  return out[:n, :c_]


# speedup vs baseline: 1.3674x; 1.3223x over previous
import functools

import jax
import jax.numpy as jnp
from jax import lax
from jax.experimental import pallas as pl
from jax.experimental.pallas import tpu as pltpu
from jax.experimental.pallas import tpu_sc as plsc

F = 16
SPC = 16
NC = 2
NW = NC * SPC
CHUNK = 128
CH2 = 2048


def _propagate_T(hsT, sidx, didx, np_, nch):
  half = nch * CH2

  mesh = plsc.VectorSubcoreMesh(core_axis_name="c", subcore_axis_name="s")

  @functools.partial(
      pl.kernel,
      mesh=mesh,
      out_type=[
          jax.ShapeDtypeStruct((F, np_), jnp.float32),
          jax.ShapeDtypeStruct((F, np_), jnp.float32),
      ],
      scratch_types=[
          pltpu.VMEM((np_,), jnp.float32),
          pltpu.VMEM((np_,), jnp.float32),
          pltpu.VMEM((2 * CH2,), jnp.int32),
          pltpu.VMEM((2 * CH2,), jnp.int32),
          pltpu.SemaphoreType.DMA,
      ],
      compiler_params=pltpu.CompilerParams(
          use_tc_tiling_on_sc=False, needs_layout_passes=False),
  )
  def k(hsT_hbm, sidx_hbm, didx_hbm, paT_hbm, pbT_hbm, ht, acc, sv, dv, sem):
    c = lax.axis_index("c")
    s = lax.axis_index("s")
    base = c * half

    tcp = pltpu.async_copy(hsT_hbm.at[s], ht, sem)

    def zbody(i, carry):
      for u in range(8):
        acc[pl.ds((i * 8 + u) * F, F)] = jnp.zeros((F,), jnp.float32)
      return carry

    lax.fori_loop(0, np_ // F // 8, zbody, 0)
    tcp.wait()

    pltpu.async_copy(sidx_hbm.at[pl.ds(base, CH2)], sv.at[pl.ds(0, CH2)], sem)
    pltpu.async_copy(didx_hbm.at[pl.ds(base, CH2)], dv.at[pl.ds(0, CH2)], sem)

    def chunk(j, carry):
      off = lax.rem(j, 2) * CH2
      nxt = CH2 - off
      pltpu.make_async_copy(
          sidx_hbm.at[pl.ds(base + j * CH2, CH2)],
          sv.at[pl.ds(off, CH2)], sem).wait()
      pltpu.make_async_copy(
          didx_hbm.at[pl.ds(base + j * CH2, CH2)],
          dv.at[pl.ds(off, CH2)], sem).wait()

      @pl.when(j + 1 < nch)
      def _():
        pltpu.async_copy(
            sidx_hbm.at[pl.ds(base + (j + 1) * CH2, CH2)],
            sv.at[pl.ds(nxt, CH2)], sem)
        pltpu.async_copy(
            didx_hbm.at[pl.ds(base + (j + 1) * CH2, CH2)],
            dv.at[pl.ds(nxt, CH2)], sem)

      @plsc.parallel_loop(0, CH2 // F, step=1, unroll=8)
      def inner(v):
        p = off + v * F
        s16 = sv[pl.ds(p, F)]
        d16 = dv[pl.ds(p, F)]
        vals = plsc.load_gather(ht, [s16])
        plsc.addupdate_scatter(acc, [d16], vals)

      return carry

    lax.fori_loop(0, nch, chunk, 0)

    @pl.when(c == 0)
    def _():
      pltpu.sync_copy(acc, paT_hbm.at[s])

    @pl.when(c == 1)
    def _():
      pltpu.sync_copy(acc, pbT_hbm.at[s])

  return k(hsT, sidx, didx)


def _degrees(didx2, np_, ch_w):
  rps = np_ // SPC

  mesh = plsc.VectorSubcoreMesh(core_axis_name="c", subcore_axis_name="s")

  @functools.partial(
      pl.kernel,
      mesh=mesh,
      out_type=[
          jax.ShapeDtypeStruct((np_,), jnp.float32),
          jax.ShapeDtypeStruct((np_,), jnp.float32),
      ],
      scratch_types=[
          pltpu.VMEM_SHARED((np_,), jnp.float32),
          pltpu.VMEM((ch_w, CHUNK), jnp.int32),
          pltpu.VMEM((rps,), jnp.float32),
      ],
      compiler_params=pltpu.CompilerParams(use_tc_tiling_on_sc=False),
  )
  def k(didx_hbm, da_hbm, db_hbm, accd, dv, buf):
    c = lax.axis_index("c")
    s = lax.axis_index("s")
    w = c * SPC + s
    r0 = s * rps
    for i in range(rps // F):
      buf[pl.ds(i * F, F)] = jnp.full((F,), 1.0, jnp.float32)
    pltpu.sync_copy(buf, accd.at[pl.ds(r0, rps)])
    pltpu.sync_copy(didx_hbm.at[pl.ds(w * ch_w, ch_w)], dv)
    plsc.subcore_barrier()

    def body(j, carry):
      pltpu.sync_copy(buf.at[pl.ds(0, CHUNK)], accd.at[dv.at[j]], add=True)
      return carry

    lax.fori_loop(0, ch_w, body, 0)

    plsc.subcore_barrier()

    @pl.when(c == 0)
    def _():
      pltpu.sync_copy(accd.at[pl.ds(r0, rps)], da_hbm.at[pl.ds(r0, rps)])

    @pl.when(c == 1)
    def _():
      pltpu.sync_copy(accd.at[pl.ds(r0, rps)], db_hbm.at[pl.ds(r0, rps)])

  return k(didx2)


def _tc_prep(da, db, x, w1, np_):
  def body(da_ref, db_ref, x_ref, w1_ref, dinv_ref, hsT_ref):
    deg = da_ref[...] + db_ref[...] - 1.0
    dinv = lax.rsqrt(deg)
    hT = lax.dot_general(
        w1_ref[...], x_ref[...],
        dimension_numbers=(((0,), (1,)), ((), ())),
        preferred_element_type=jnp.float32)
    dinv_ref[...] = dinv
    hsT_ref[...] = dinv * hT

  return pl.pallas_call(
      body,
      out_shape=[
          jax.ShapeDtypeStruct((1, np_), jnp.float32),
          jax.ShapeDtypeStruct((F, np_), jnp.float32),
      ],
  )(da, db, x, w1)


def _tc_layer(dinv, paT, pbT, hsT, b1c, w2pT, np_):
  def body(dinv_ref, pa_ref, pb_ref, hsp_ref, b1_ref, w2t_ref, out_ref):
    t = dinv_ref[...] * (pa_ref[...] + pb_ref[...] + hsp_ref[...]) + b1_ref[...]
    h2 = jnp.maximum(t, 0.0)
    out_ref[...] = dinv_ref[...] * jnp.dot(
        w2t_ref[...], h2, preferred_element_type=jnp.float32)

  return pl.pallas_call(
      body,
      out_shape=jax.ShapeDtypeStruct((F, np_), jnp.float32),
  )(dinv, paT, pbT, hsT, b1c, w2pT)


def _tc_final(dinv, qaT, qbT, hs2T, b2c, np_, c_):
  def body(dinv_ref, qa_ref, qb_ref, hsp_ref, b2_ref, out_ref):
    t = dinv_ref[...] * (qa_ref[...] + qb_ref[...] + hsp_ref[...]) + b2_ref[...]
    r = jnp.maximum(t, 0.0)
    row = lax.broadcasted_iota(jnp.int32, (F, np_), 0)
    valid = row < c_
    rm = jnp.where(valid, r, jnp.float32(-1e30))
    m = jnp.max(rm, axis=0, keepdims=True)
    e = jnp.where(valid, jnp.exp(rm - m), 0.0)
    ssum = jnp.sum(e, axis=0, keepdims=True)
    res = rm - m - jnp.log(ssum)
    out_ref[...] = res.T

  return pl.pallas_call(
      body,
      out_shape=jax.ShapeDtypeStruct((np_, F), jnp.float32),
  )(dinv, qaT, qbT, hs2T, b2c)


def kernel(x, edge_index, W1, b1, W2, b2):
  n, d = x.shape
  h = W1.shape[1]
  c_ = W2.shape[1]
  assert h == F
  e = edge_index.shape[1]

  np_ = ((n + 255) // 256) * 256
  nch = -(-e // (NC * CH2))
  ep = NC * nch * CH2
  ch_w = ep // (NW * CHUNK)

  src = edge_index[0]
  dst = edge_index[1]
  pad = jnp.full((ep - e,), n, dtype=jnp.int32)
  sidx = jnp.concatenate([src, pad])
  didx = jnp.concatenate([dst, pad])

  xp = jnp.pad(x, ((0, np_ - n), (0, 0)))

  da, db = _degrees(didx.reshape(NW * ch_w, CHUNK), np_, ch_w)
  dinv, hsT = _tc_prep(da.reshape(1, np_), db.reshape(1, np_), xp, W1, np_)
  paT, pbT = _propagate_T(hsT, sidx, didx, np_, nch)
  w2pT = jnp.pad(W2, ((0, 0), (0, F - c_))).T
  hs2T = _tc_layer(dinv, paT, pbT, hsT, b1.reshape(F, 1), w2pT, np_)
  qaT, qbT = _propagate_T(hs2T, sidx, didx, np_, nch)
  b2c = jnp.pad(b2, (0, F - c_)).reshape(F, 1)
  out = _tc_final(dinv, qaT, qbT, hs2T, b2c, np_, c_)
  return out[:n, :c_]
